# baseline (speedup 1.0000x reference)
import functools

import jax
import jax.numpy as jnp
from jax import lax
from jax.experimental import pallas as pl
from jax.experimental.pallas import tpu as pltpu
from jax.experimental.pallas import tpu_sc as plsc

D_MODEL = 128


@functools.cache
def _make_gather(num_rows: int, d: int, total: int, chunk: int):
    info = plsc.get_sparse_core_info()
    nw = info.num_cores * info.num_subcores
    assert total % nw == 0
    b_per_w = total // nw
    assert b_per_w % chunk == 0
    n_chunks = b_per_w // chunk
    assert n_chunks % 2 == 0
    mesh = plsc.VectorSubcoreMesh(core_axis_name="c", subcore_axis_name="s")

    @functools.partial(
        pl.kernel,
        mesh=mesh,
        out_type=jax.ShapeDtypeStruct((total, d), jnp.float32),
        scratch_types=[
            pltpu.VMEM((b_per_w,), jnp.int32),
            pltpu.VMEM((chunk, d), jnp.float32),
            pltpu.VMEM((chunk, d), jnp.float32),
            pltpu.SemaphoreType.DMA,
            pltpu.SemaphoreType.DMA,
            pltpu.SemaphoreType.DMA,
        ],
    )
    def gather_kernel(table_hbm, idx_hbm, out_hbm,
                      idx_v, rows0, rows1, sem_g, st0, st1):
        rows_v = (rows0, rows1)
        st = (st0, st1)
        wid = lax.axis_index("s") * info.num_cores + lax.axis_index("c")
        base = wid * b_per_w

        pltpu.sync_copy(idx_hbm.at[pl.ds(base, b_per_w)], idx_v)

        def pair_body(j, carry):
            for k in range(2):
                i = 2 * j + k
                rb, sst = rows_v[k], st[k]

                @pl.when(j > 0)
                def _wait_store():
                    pltpu.make_async_copy(
                        rb, out_hbm.at[pl.ds(base, chunk)], sst).wait()

                pltpu.async_copy(
                    table_hbm.at[idx_v.at[pl.ds(i * chunk, chunk)]],
                    rb, sem_g).wait()

                pltpu.async_copy(
                    rb, out_hbm.at[pl.ds(base + i * chunk, chunk)], sst)
            return carry

        lax.fori_loop(0, n_chunks // 2, pair_body, 0)

        pltpu.make_async_copy(rows0, out_hbm.at[pl.ds(base, chunk)], st0).wait()
        pltpu.make_async_copy(rows1, out_hbm.at[pl.ds(base, chunk)], st1).wait()

    return gather_kernel


def kernel(new_token_ids, new_emb_weight):
    b, h = new_token_ids.shape
    v, d = new_emb_weight.shape
    idx = new_token_ids.reshape(-1).astype(jnp.int32)
    out = _make_gather(v, d, b * h, 400)(new_emb_weight, idx)
    return out.reshape(b, h, d)


# speedup vs baseline: 1.0068x; 1.0068x over previous
import functools

import jax
import jax.numpy as jnp
from jax import lax
from jax.experimental import pallas as pl
from jax.experimental.pallas import tpu as pltpu
from jax.experimental.pallas import tpu_sc as plsc

D_MODEL = 128
NBUF = 4


@functools.cache
def _make_gather(num_rows: int, d: int, total: int, chunk: int):
    info = plsc.get_sparse_core_info()
    nw = info.num_cores * info.num_subcores
    assert total % nw == 0
    b_per_w = total // nw
    assert b_per_w % chunk == 0
    n_chunks = b_per_w // chunk
    assert n_chunks % NBUF == 0 and n_chunks >= 2 * NBUF
    mesh = plsc.VectorSubcoreMesh(core_axis_name="c", subcore_axis_name="s")

    scratch = (
        [pltpu.VMEM((b_per_w,), jnp.int32)]
        + [pltpu.VMEM((chunk, d), jnp.float32) for _ in range(NBUF)]
        + [pltpu.SemaphoreType.DMA for _ in range(2 * NBUF)]
    )

    @functools.partial(
        pl.kernel,
        mesh=mesh,
        out_type=jax.ShapeDtypeStruct((total, d), jnp.float32),
        scratch_types=scratch,
    )
    def gather_kernel(table_hbm, idx_hbm, out_hbm, idx_v, *bufs):
        rows_v = bufs[:NBUF]
        sem_g = bufs[NBUF:2 * NBUF]
        st = bufs[2 * NBUF:3 * NBUF]
        wid = lax.axis_index("s") * info.num_cores + lax.axis_index("c")
        base = wid * b_per_w

        def start_gather(i, b):
            pltpu.async_copy(
                table_hbm.at[idx_v.at[pl.ds(i * chunk, chunk)]],
                rows_v[b], sem_g[b])

        def wait_gather(b):
            pltpu.make_async_copy(
                table_hbm.at[idx_v.at[pl.ds(0, chunk)]],
                rows_v[b], sem_g[b]).wait()

        def start_store(i, b):
            pltpu.async_copy(
                rows_v[b], out_hbm.at[pl.ds(base + i * chunk, chunk)], st[b])

        def wait_store(b):
            pltpu.make_async_copy(
                rows_v[b], out_hbm.at[pl.ds(base, chunk)], st[b]).wait()

        pltpu.sync_copy(idx_hbm.at[pl.ds(base, b_per_w)], idx_v)
        start_gather(0, 0)

        def quad_body(j, carry):
            for k in range(NBUF):
                i = NBUF * j + k
                b = k
                p = (k - 1) % NBUF

                @pl.when(i > 0)
                def _advance():
                    @pl.when(i >= NBUF)
                    def _():
                        wait_store(b)

                    start_gather(i, b)
                    wait_gather(p)
                    start_store(i - 1, p)
            return carry

        lax.fori_loop(0, n_chunks // NBUF, quad_body, 0)

        last = n_chunks - 1
        lb = last % NBUF
        wait_gather(lb)
        start_store(last, lb)
        for b in range(NBUF):
            wait_store(b)

    return gather_kernel


def kernel(new_token_ids, new_emb_weight):
    b, h = new_token_ids.shape
    v, d = new_emb_weight.shape
    idx = new_token_ids.reshape(-1).astype(jnp.int32)
    out = _make_gather(v, d, b * h, 200)(new_emb_weight, idx)
    return out.reshape(b, h, d)


# speedup vs baseline: 1.0093x; 1.0025x over previous
import functools

import jax
import jax.numpy as jnp
from jax import lax
from jax.experimental import pallas as pl
from jax.experimental.pallas import tpu as pltpu
from jax.experimental.pallas import tpu_sc as plsc

D_MODEL = 128
NBUF = 4


@functools.cache
def _make_gather(num_rows: int, d: int, total: int, chunk: int):
    info = plsc.get_sparse_core_info()
    nw = info.num_cores * info.num_subcores
    assert total % nw == 0
    b_per_w = total // nw
    assert b_per_w % chunk == 0
    n_chunks = b_per_w // chunk
    assert n_chunks % NBUF == 0 and n_chunks >= 2 * NBUF
    mesh = plsc.VectorSubcoreMesh(core_axis_name="c", subcore_axis_name="s")

    scratch = (
        [pltpu.VMEM((chunk,), jnp.int32) for _ in range(NBUF)]
        + [pltpu.VMEM((chunk, d), jnp.float32) for _ in range(NBUF)]
        + [pltpu.SemaphoreType.DMA for _ in range(3 * NBUF)]
    )

    @functools.partial(
        pl.kernel,
        mesh=mesh,
        out_type=jax.ShapeDtypeStruct((total, d), jnp.float32),
        scratch_types=scratch,
    )
    def gather_kernel(table_hbm, idx_hbm, out_hbm, *bufs):
        idx_v = bufs[:NBUF]
        rows_v = bufs[NBUF:2 * NBUF]
        sem_g = bufs[2 * NBUF:3 * NBUF]
        st = bufs[3 * NBUF:4 * NBUF]
        si = bufs[4 * NBUF:5 * NBUF]
        wid = lax.axis_index("s") * info.num_cores + lax.axis_index("c")
        base = wid * b_per_w

        def wait_idx(b):
            pltpu.make_async_copy(
                idx_hbm.at[pl.ds(base, chunk)], idx_v[b], si[b]).wait()

        def wait_store(b):
            pltpu.make_async_copy(
                rows_v[b], out_hbm.at[pl.ds(base, chunk)], st[b]).wait()

        def wait_gather(b):
            pltpu.make_async_copy(
                table_hbm.at[idx_v[b]], rows_v[b], sem_g[b]).wait()

        for b in range(NBUF):
            pltpu.async_copy(
                idx_hbm.at[pl.ds(base + b * chunk, chunk)], idx_v[b], si[b])

        wait_idx(0)
        pltpu.async_copy(table_hbm.at[idx_v[0]], rows_v[0], sem_g[0])

        def quad_body(j, carry):
            for k in range(NBUF):
                i = NBUF * j + k
                b = k
                p = (k - 1) % NBUF

                @pl.when(i > 0)
                def _advance():
                    wait_idx(b)

                    @pl.when(i >= NBUF)
                    def _():
                        wait_store(b)

                    pltpu.async_copy(table_hbm.at[idx_v[b]], rows_v[b],
                                     sem_g[b])
                    wait_gather(p)

                    @pl.when(i + NBUF - 1 < n_chunks)
                    def _():
                        pltpu.async_copy(
                            idx_hbm.at[
                                pl.ds(base + (i + NBUF - 1) * chunk, chunk)],
                            idx_v[p], si[p])

                    pltpu.async_copy(
                        rows_v[p], out_hbm.at[pl.ds(base + (i - 1) * chunk,
                                                    chunk)], st[p])
            return carry

        lax.fori_loop(0, n_chunks // NBUF, quad_body, 0)

        last = n_chunks - 1
        lb = last % NBUF
        wait_gather(lb)
        pltpu.async_copy(rows_v[lb], out_hbm.at[pl.ds(base + last * chunk,
                                                      chunk)], st[lb])
        for b in range(NBUF):
            wait_store(b)

    return gather_kernel


def kernel(new_token_ids, new_emb_weight):
    b, h = new_token_ids.shape
    v, d = new_emb_weight.shape
    idx = new_token_ids.reshape(-1).astype(jnp.int32)
    out = _make_gather(v, d, b * h, 200)(new_emb_weight, idx)
    return out.reshape(b, h, d)


# speedup vs baseline: 1.0258x; 1.0163x over previous
import functools

import jax
import jax.numpy as jnp
from jax import lax
from jax.experimental import pallas as pl
from jax.experimental.pallas import tpu as pltpu
from jax.experimental.pallas import tpu_sc as plsc

D_MODEL = 128
NBUF = 4


@functools.cache
def _make_gather(num_rows: int, d: int, total: int, chunk: int):
    info = plsc.get_sparse_core_info()
    nw = info.num_cores * info.num_subcores
    assert total % nw == 0
    b_per_w = total // nw
    assert b_per_w % chunk == 0
    n_chunks = b_per_w // chunk
    assert n_chunks % NBUF == 0 and n_chunks >= 2 * NBUF
    mesh = plsc.VectorSubcoreMesh(core_axis_name="c", subcore_axis_name="s")

    scratch = (
        [pltpu.VMEM((chunk,), jnp.int32) for _ in range(NBUF)]
        + [pltpu.VMEM((chunk, d), jnp.float32) for _ in range(NBUF)]
        + [pltpu.SemaphoreType.DMA for _ in range(3 * NBUF)]
    )

    @functools.partial(
        pl.kernel,
        mesh=mesh,
        out_type=jax.ShapeDtypeStruct((total, d), jnp.float32),
        scratch_types=scratch,
    )
    def gather_kernel(table_hbm, idx_hbm, out_hbm, *bufs):
        idx_v = bufs[:NBUF]
        rows_v = bufs[NBUF:2 * NBUF]
        sem_g = bufs[2 * NBUF:3 * NBUF]
        st = bufs[3 * NBUF:4 * NBUF]
        si = bufs[4 * NBUF:5 * NBUF]
        wid = lax.axis_index("s") * info.num_cores + lax.axis_index("c")
        base = wid * b_per_w

        row0 = wid * (b_per_w // chunk)

        def wait_idx(b):
            pltpu.make_async_copy(
                idx_hbm.at[0], idx_v[b], si[b]).wait()

        def wait_store(b):
            pltpu.make_async_copy(
                rows_v[b], out_hbm.at[pl.ds(base, chunk)], st[b]).wait()

        def wait_gather(b):
            pltpu.make_async_copy(
                table_hbm.at[idx_v[b]], rows_v[b], sem_g[b]).wait()

        for b in range(NBUF):
            pltpu.async_copy(idx_hbm.at[row0 + b], idx_v[b], si[b])

        wait_idx(0)
        pltpu.async_copy(table_hbm.at[idx_v[0]], rows_v[0], sem_g[0])

        def quad_body(j, carry):
            for k in range(NBUF):
                i = NBUF * j + k
                b = k
                p = (k - 1) % NBUF

                @pl.when(i > 0)
                def _advance():
                    wait_idx(b)

                    @pl.when(i >= NBUF)
                    def _():
                        wait_store(b)

                    pltpu.async_copy(table_hbm.at[idx_v[b]], rows_v[b],
                                     sem_g[b])
                    wait_gather(p)

                    @pl.when(i + NBUF - 1 < n_chunks)
                    def _():
                        pltpu.async_copy(
                            idx_hbm.at[row0 + i + NBUF - 1], idx_v[p], si[p])

                    pltpu.async_copy(
                        rows_v[p], out_hbm.at[pl.ds(base + (i - 1) * chunk,
                                                    chunk)], st[p])
            return carry

        lax.fori_loop(0, n_chunks // NBUF, quad_body, 0)

        last = n_chunks - 1
        lb = last % NBUF
        wait_gather(lb)
        pltpu.async_copy(rows_v[lb], out_hbm.at[pl.ds(base + last * chunk,
                                                      chunk)], st[lb])
        for b in range(NBUF):
            wait_store(b)

    return gather_kernel


def kernel(new_token_ids, new_emb_weight):
    b, h = new_token_ids.shape
    v, d = new_emb_weight.shape
    idx = new_token_ids.astype(jnp.int32)
    out = _make_gather(v, d, b * h, h)(new_emb_weight, idx)
    return out.reshape(b, h, d)


# speedup vs baseline: 1.0269x; 1.0010x over previous
import functools

import jax
import jax.numpy as jnp
from jax import lax
from jax.experimental import pallas as pl
from jax.experimental.pallas import tpu as pltpu
from jax.experimental.pallas import tpu_sc as plsc

D_MODEL = 128
NBUF = 4


@functools.cache
def _make_gather(num_rows: int, d: int, total: int, chunk: int):
    info = plsc.get_sparse_core_info()
    nw = info.num_cores * info.num_subcores
    assert total % nw == 0
    b_per_w = total // nw
    assert b_per_w % chunk == 0
    n_chunks = b_per_w // chunk
    assert n_chunks % NBUF == 0 and n_chunks >= 2 * NBUF
    mesh = plsc.VectorSubcoreMesh(core_axis_name="c", subcore_axis_name="s")

    scratch = (
        [pltpu.VMEM((chunk,), jnp.int32) for _ in range(NBUF)]
        + [pltpu.VMEM((chunk, d), jnp.float32) for _ in range(NBUF)]
        + [pltpu.SemaphoreType.DMA for _ in range(3 * NBUF)]
    )

    @functools.partial(
        pl.kernel,
        mesh=mesh,
        out_type=jax.ShapeDtypeStruct((total // chunk, chunk, d),
                                      jnp.float32),
        scratch_types=scratch,
    )
    def gather_kernel(table_hbm, idx_hbm, out_hbm, *bufs):
        idx_v = bufs[:NBUF]
        rows_v = bufs[NBUF:2 * NBUF]
        sem_g = bufs[2 * NBUF:3 * NBUF]
        st = bufs[3 * NBUF:4 * NBUF]
        si = bufs[4 * NBUF:5 * NBUF]
        wid = lax.axis_index("s") * info.num_cores + lax.axis_index("c")
        row0 = wid * (b_per_w // chunk)

        def wait_idx(b):
            pltpu.make_async_copy(
                idx_hbm.at[0], idx_v[b], si[b]).wait()

        def wait_store(b):
            pltpu.make_async_copy(rows_v[b], out_hbm.at[0], st[b]).wait()

        def wait_gather(b):
            pltpu.make_async_copy(
                table_hbm.at[idx_v[b]], rows_v[b], sem_g[b]).wait()

        for b in range(NBUF):
            pltpu.async_copy(idx_hbm.at[row0 + b], idx_v[b], si[b])

        wait_idx(0)
        pltpu.async_copy(table_hbm.at[idx_v[0]], rows_v[0], sem_g[0])

        def quad_body(j, carry):
            for k in range(NBUF):
                i = NBUF * j + k
                b = k
                p = (k - 1) % NBUF

                @pl.when(i > 0)
                def _advance():
                    wait_idx(b)

                    @pl.when(i >= NBUF)
                    def _():
                        wait_store(b)

                    pltpu.async_copy(table_hbm.at[idx_v[b]], rows_v[b],
                                     sem_g[b])
                    wait_gather(p)

                    @pl.when(i + NBUF - 1 < n_chunks)
                    def _():
                        pltpu.async_copy(
                            idx_hbm.at[row0 + i + NBUF - 1], idx_v[p], si[p])

                    pltpu.async_copy(rows_v[p], out_hbm.at[row0 + i - 1],
                                     st[p])
            return carry

        lax.fori_loop(0, n_chunks // NBUF, quad_body, 0)

        last = n_chunks - 1
        lb = last % NBUF
        wait_gather(lb)
        pltpu.async_copy(rows_v[lb], out_hbm.at[row0 + last], st[lb])
        for b in range(NBUF):
            wait_store(b)

    return gather_kernel


def kernel(new_token_ids, new_emb_weight):
    b, h = new_token_ids.shape
    v, d = new_emb_weight.shape
    idx = new_token_ids.astype(jnp.int32)
    return _make_gather(v, d, b * h, h)(new_emb_weight, idx)


# speedup vs baseline: 1.0269x; 1.0001x over previous
import functools

import jax
import jax.numpy as jnp
from jax import lax
from jax.experimental import pallas as pl
from jax.experimental.pallas import tpu as pltpu
from jax.experimental.pallas import tpu_sc as plsc

NBUF = 4


@functools.cache
def _make_gather(num_rows: int, d: int, total: int, chunk: int):
    info = plsc.get_sparse_core_info()
    nw = info.num_cores * info.num_subcores
    assert total % nw == 0
    b_per_w = total // nw
    assert b_per_w % chunk == 0
    n_chunks = b_per_w // chunk
    assert n_chunks % NBUF == 0 and n_chunks >= 2 * NBUF
    mesh = plsc.VectorSubcoreMesh(core_axis_name="c", subcore_axis_name="s")

    scratch = (
        [pltpu.VMEM((chunk,), jnp.int32) for _ in range(NBUF)]
        + [pltpu.VMEM((chunk, d), jnp.float32) for _ in range(NBUF)]
        + [pltpu.SemaphoreType.DMA for _ in range(3 * NBUF)]
    )

    @functools.partial(
        pl.kernel,
        mesh=mesh,
        out_type=jax.ShapeDtypeStruct((total // chunk, chunk, d),
                                      jnp.float32),
        scratch_types=scratch,
    )
    def gather_kernel(table_hbm, idx_hbm, out_hbm, *bufs):
        idx_v = bufs[:NBUF]
        rows_v = bufs[NBUF:2 * NBUF]
        sem_g = bufs[2 * NBUF:3 * NBUF]
        st = bufs[3 * NBUF:4 * NBUF]
        si = bufs[4 * NBUF:5 * NBUF]
        wid = lax.axis_index("s") * info.num_cores + lax.axis_index("c")
        row0 = wid * (b_per_w // chunk)

        def wait_idx(b):
            pltpu.make_async_copy(
                idx_hbm.at[0], idx_v[b], si[b]).wait()

        def wait_store(b):
            pltpu.make_async_copy(rows_v[b], out_hbm.at[0], st[b]).wait()

        def wait_gather(b):
            pltpu.make_async_copy(
                table_hbm.at[idx_v[b]], rows_v[b], sem_g[b]).wait()

        for b in range(NBUF):
            pltpu.async_copy(idx_hbm.at[row0 + b], idx_v[b], si[b])

        wait_idx(0)
        pltpu.async_copy(table_hbm.at[idx_v[0]], rows_v[0], sem_g[0])

        def quad_body(j, carry):
            for k in range(NBUF):
                i = NBUF * j + k
                b = k
                p = (k - 1) % NBUF

                @pl.when(i > 0)
                def _advance():
                    wait_idx(b)

                    @pl.when(i >= NBUF)
                    def _():
                        wait_store(b)

                    pltpu.async_copy(table_hbm.at[idx_v[b]], rows_v[b],
                                     sem_g[b])
                    wait_gather(p)

                    @pl.when(i + NBUF - 1 < n_chunks)
                    def _():
                        pltpu.async_copy(
                            idx_hbm.at[row0 + i + NBUF - 1], idx_v[p], si[p])

                    pltpu.async_copy(rows_v[p], out_hbm.at[row0 + i - 1],
                                     st[p])
            return carry

        lax.fori_loop(0, n_chunks // NBUF, quad_body, 0)

        last = n_chunks - 1
        lb = last % NBUF
        wait_gather(lb)
        pltpu.async_copy(rows_v[lb], out_hbm.at[row0 + last], st[lb])
        for b in range(NBUF):
            wait_store(b)

    return gather_kernel


def kernel(new_token_ids, new_emb_weight):
    b, h = new_token_ids.shape
    v, d = new_emb_weight.shape
    idx = new_token_ids.astype(jnp.int32)
    return _make_gather(v, d, b * h, h)(new_emb_weight, idx)
